# baseline (speedup 1.0000x reference)
import functools

import jax
import jax.numpy as jnp
from jax import lax
from jax.experimental import pallas as pl
from jax.experimental.pallas import tpu as pltpu
from jax.experimental.pallas import tpu_sc as plsc

_N = 10000
_R = 4
_E = 80000
_D = 128
_NH = 10240
_CW = 80
_TILES = 16
_CPT = 64
_NCHUNK = _CPT * _TILES
_EPAD = _NCHUNK * _CW
_RPT = _NH // _TILES

_sc_mesh = plsc.VectorSubcoreMesh(core_axis_name="c", subcore_axis_name="s")
_sc_params = pltpu.CompilerParams(needs_layout_passes=False)


@functools.partial(
    pl.kernel, mesh=_sc_mesh, compiler_params=_sc_params,
    out_type=jax.ShapeDtypeStruct((_R, 2, _TILES, _NH), jnp.float32),
    scratch_types=[
        pltpu.VMEM((_CPT, _CW), jnp.int32),
        pltpu.VMEM((_NH,), jnp.float32),
    ],
)
def _degree_kernel(ei_hbm, cnt_hbm, idxbuf, hist):
    cid = lax.axis_index("c")
    tid = lax.axis_index("s")
    ones = jnp.ones((16,), jnp.float32)
    zeros = jnp.zeros((16,), jnp.float32)
    for ri in range(_R // 2):
        r = cid * (_R // 2) + ri
        for side in range(2):
            pltpu.sync_copy(ei_hbm.at[r, side, pl.ds(tid * _CPT, _CPT), :],
                            idxbuf)

            def zero_body(j, carry):
                for l in range(16):
                    hist[pl.ds(j * 256 + l * 16, 16)] = zeros
                return carry

            lax.fori_loop(0, _NH // 256, zero_body, 0)

            def acc_body(j, carry):
                for l in range(_CW // 16):
                    idx = idxbuf[j, pl.ds(l * 16, 16)]
                    plsc.addupdate_scatter(hist, [idx], ones)
                return carry

            lax.fori_loop(0, _CPT, acc_body, 0)
            pltpu.sync_copy(hist, cnt_hbm.at[r, side, tid])


_BLK = 512


def _mm_body(x_ref, w_ref, y_ref):
    x = x_ref[...]
    for r in range(_R):
        y_ref[r] = jnp.dot(x, w_ref[r], preferred_element_type=jnp.float32)


def _mm(x_pad, w):
    return pl.pallas_call(
        _mm_body,
        grid=(_NH // _BLK,),
        in_specs=[
            pl.BlockSpec((_BLK, _D), lambda i: (i, 0)),
            pl.BlockSpec((_R, _D, _D), lambda i: (0, 0, 0)),
        ],
        out_specs=pl.BlockSpec((_R, _BLK, _D), lambda i: (0, i, 0)),
        out_shape=jax.ShapeDtypeStruct((_R, _NH, _D), jnp.float32),
    )(x_pad, w)


def _scale_body(y_ref, cnt_ref, z_ref):
    for r in range(_R):
        cnt = jnp.sum(cnt_ref[r, 0], axis=0)
        norm = lax.rsqrt(jnp.maximum(cnt, 1.0))
        z_ref[r] = y_ref[r] * norm[:, None]


def _scale(y, cnt):
    return pl.pallas_call(
        _scale_body,
        grid=(_NH // _BLK,),
        in_specs=[
            pl.BlockSpec((_R, _BLK, _D), lambda i: (0, i, 0)),
            pl.BlockSpec((_R, 1, _TILES, _BLK), lambda i: (0, 0, 0, i)),
        ],
        out_specs=pl.BlockSpec((_R, _BLK, _D), lambda i: (0, i, 0)),
        out_shape=jax.ShapeDtypeStruct((_R, _NH, _D), jnp.float32),
    )(y, cnt)


@functools.partial(
    pl.kernel, mesh=_sc_mesh, compiler_params=_sc_params,
    out_type=jax.ShapeDtypeStruct((_R, _NH, _D), jnp.float32),
    scratch_types=[
        pltpu.VMEM((_CPT + 2, _CW), jnp.int32),
        pltpu.VMEM((_CPT, _CW), jnp.int32),
        pltpu.VMEM((_CW, _D), jnp.float32),
        pltpu.VMEM((_CW, _D), jnp.float32),
        pltpu.VMEM((_CW, _D), jnp.float32),
        pltpu.VMEM_SHARED((_NH, _D), jnp.float32),
        pltpu.SemaphoreType.DMA,
        pltpu.SemaphoreType.DMA,
        pltpu.SemaphoreType.DMA,
        pltpu.SemaphoreType.DMA,
        pltpu.SemaphoreType.DMA,
        pltpu.SemaphoreType.DMA,
    ],
)
def _edge_kernel(z_hbm, src_hbm, dst_hbm, agg_hbm,
                 srcbuf, dstbuf, rows0, rows1, rows2, aggsh,
                 g0, g1, g2, s0, s1, s2):
    cid = lax.axis_index("c")
    tid = lax.axis_index("s")
    zeros = jnp.zeros((16,), jnp.float32)
    rows = (rows0, rows1, rows2)
    gsem = (g0, g1, g2)
    ssem = (s0, s1, s2)

    def wait_gather(k):
        pltpu.make_async_copy(z_hbm.at[srcbuf.at[0]], rows[k],
                              gsem[k]).wait()

    def wait_scatter(k):
        pltpu.make_async_copy(rows[k], aggsh.at[dstbuf.at[0]],
                              ssem[k]).wait()

    for ri in range(_R // 2):
        r = cid * (_R // 2) + ri
        roff = r * _NH

        def zb_body(j, carry):
            for l in range(8):
                rows0[j, pl.ds(l * 16, 16)] = zeros
            return carry

        lax.fori_loop(0, _CW, zb_body, 0)
        for p in range(_RPT // _CW):
            pltpu.sync_copy(
                rows0, aggsh.at[pl.ds(tid * _RPT + p * _CW, _CW), :])
        plsc.subcore_barrier()

        pltpu.sync_copy(src_hbm.at[r, pl.ds(tid * _CPT, _CPT), :],
                        srcbuf.at[pl.ds(0, _CPT), :])
        pltpu.sync_copy(dst_hbm.at[r, pl.ds(tid * _CPT, _CPT), :], dstbuf)

        def off_body(j, carry):
            for l in range(_CW // 16):
                sl = pl.ds(l * 16, 16)
                srcbuf[j, sl] = srcbuf[j, sl] + roff
            return carry

        lax.fori_loop(0, _CPT, off_body, 0)
        pad = roff + _N + lax.iota(jnp.int32, 16)
        for j in (_CPT, _CPT + 1):
            for l in range(_CW // 16):
                srcbuf[j, pl.ds(l * 16, 16)] = pad + l * 16

        def step(c, k, first):
            wait_gather(k)
            pltpu.async_copy(rows[k], aggsh.at[dstbuf.at[c]], ssem[k],
                             add=True)
            k2 = (k + 2) % 3
            if not first:
                wait_scatter(k2)
            pltpu.async_copy(z_hbm.at[srcbuf.at[c + 2]], rows[k2], gsem[k2])

        pltpu.async_copy(z_hbm.at[srcbuf.at[0]], rows[0], gsem[0])
        pltpu.async_copy(z_hbm.at[srcbuf.at[1]], rows[1], gsem[1])
        step(0, 0, True)
        step(1, 1, False)
        step(2, 2, False)

        def chunk_body(jj, carry):
            c = jj * 3
            step(c, 0, False)
            step(c + 1, 1, False)
            step(c + 2, 2, False)
            return carry

        lax.fori_loop(1, 21, chunk_body, 0)
        step(63, 0, False)
        wait_scatter((_CPT - 1) % 3)
        wait_gather(_CPT % 3)
        wait_gather((_CPT + 1) % 3)
        plsc.subcore_barrier()
        pltpu.sync_copy(aggsh.at[pl.ds(tid * _RPT, _RPT), :],
                        agg_hbm.at[r, pl.ds(tid * _RPT, _RPT), :])
        plsc.subcore_barrier()


_BLK2 = 1024


def _out_body(agg_ref, cnt_ref, b_ref, o_ref):
    acc = jnp.broadcast_to(jnp.sum(b_ref[...], axis=0)[None, :], (_BLK2, _D))
    for r in range(_R):
        cnt = jnp.sum(cnt_ref[r, 0], axis=0)
        norm = lax.rsqrt(jnp.maximum(cnt, 1.0))
        acc = acc + agg_ref[r] * norm[:, None]
    o_ref[...] = acc


def _finalize(agg, cnt, b):
    return pl.pallas_call(
        _out_body,
        grid=(_NH // _BLK2,),
        in_specs=[
            pl.BlockSpec((_R, _BLK2, _D), lambda i: (0, i, 0)),
            pl.BlockSpec((_R, 1, _TILES, _BLK2), lambda i: (0, 1, 0, i)),
            pl.BlockSpec((_R, _D), lambda i: (0, 0)),
        ],
        out_specs=pl.BlockSpec((_BLK2, _D), lambda i: (i, 0)),
        out_shape=jax.ShapeDtypeStruct((_NH, _D), jnp.float32),
    )(agg, cnt, b)


def kernel(features, edge_index, W, b):
    x_pad = jnp.concatenate(
        [features, jnp.zeros((_NH - _N, _D), jnp.float32)], axis=0)
    fill = _N + (jnp.arange(_EPAD - _E, dtype=jnp.int32) % 128)
    fill = jnp.broadcast_to(fill[None, :], (_R, _EPAD - _E))
    src_pad = jnp.concatenate(
        [edge_index[:, 0, :], fill], axis=1).reshape(_R, _NCHUNK, _CW)
    dst_pad = jnp.concatenate(
        [edge_index[:, 1, :], fill], axis=1).reshape(_R, _NCHUNK, _CW)
    ei_pad = jnp.stack([src_pad, dst_pad], axis=1)

    y = _mm(x_pad, W)
    cnt = _degree_kernel(ei_pad)
    z = _scale(y, cnt)
    agg = _edge_kernel(z.reshape(_R * _NH, _D), src_pad, dst_pad)
    return _finalize(agg, cnt, b)[:_N]


# speedup vs baseline: 1.0809x; 1.0809x over previous
import functools

import jax
import jax.numpy as jnp
from jax import lax
from jax.experimental import pallas as pl
from jax.experimental.pallas import tpu as pltpu
from jax.experimental.pallas import tpu_sc as plsc

_N = 10000
_R = 4
_E = 80000
_D = 128
_NH = 10240
_CW = 80
_TILES = 16
_CPT = 64
_NCHUNK = _CPT * _TILES
_EPAD = _NCHUNK * _CW
_RPT = _NH // _TILES

_sc_mesh = plsc.VectorSubcoreMesh(core_axis_name="c", subcore_axis_name="s")
_sc_params = pltpu.CompilerParams(needs_layout_passes=False)


@functools.partial(
    pl.kernel, mesh=_sc_mesh, compiler_params=_sc_params,
    out_type=jax.ShapeDtypeStruct((_R, 2, _TILES, _NH), jnp.float32),
    scratch_types=[
        pltpu.VMEM((_CPT, _CW), jnp.int32),
        pltpu.VMEM((_NH,), jnp.float32),
    ],
)
def _degree_kernel(ei_hbm, cnt_hbm, idxbuf, hist):
    cid = lax.axis_index("c")
    tid = lax.axis_index("s")
    ones = jnp.ones((16,), jnp.float32)
    zeros = jnp.zeros((16,), jnp.float32)
    for ri in range(_R // 2):
        r = cid * (_R // 2) + ri
        for side in range(2):
            pltpu.sync_copy(ei_hbm.at[r, side, pl.ds(tid * _CPT, _CPT), :],
                            idxbuf)

            def zero_body(j, carry):
                for l in range(16):
                    hist[pl.ds(j * 256 + l * 16, 16)] = zeros
                return carry

            lax.fori_loop(0, _NH // 256, zero_body, 0)

            def acc_body(j, carry):
                for l in range(_CW // 16):
                    idx = idxbuf[j, pl.ds(l * 16, 16)]
                    plsc.addupdate_scatter(hist, [idx], ones)
                return carry

            lax.fori_loop(0, _CPT, acc_body, 0)
            pltpu.sync_copy(hist, cnt_hbm.at[r, side, tid])


_BLK = 512


def _zmm_body(x_ref, cnt_ref, w_ref, z_ref):
    x = x_ref[...]
    for r in range(_R):
        cnt = jnp.sum(cnt_ref[r, 0], axis=0)
        norm = lax.rsqrt(jnp.maximum(cnt, 1.0))
        z_ref[r] = jnp.dot(x * norm[:, None], w_ref[r],
                           preferred_element_type=jnp.float32)


def _zmm(x_pad, cnt, w):
    return pl.pallas_call(
        _zmm_body,
        grid=(_NH // _BLK,),
        in_specs=[
            pl.BlockSpec((_BLK, _D), lambda i: (i, 0)),
            pl.BlockSpec((_R, 1, _TILES, _BLK), lambda i: (0, 0, 0, i)),
            pl.BlockSpec((_R, _D, _D), lambda i: (0, 0, 0)),
        ],
        out_specs=pl.BlockSpec((_R, _BLK, _D), lambda i: (0, i, 0)),
        out_shape=jax.ShapeDtypeStruct((_R, _NH, _D), jnp.float32),
    )(x_pad, cnt, w)


@functools.partial(
    pl.kernel, mesh=_sc_mesh, compiler_params=_sc_params,
    out_type=jax.ShapeDtypeStruct((_R, _NH, _D), jnp.float32),
    scratch_types=[
        pltpu.VMEM((_CPT + 2, _CW), jnp.int32),
        pltpu.VMEM((_CPT, _CW), jnp.int32),
        pltpu.VMEM((_CW, _D), jnp.float32),
        pltpu.VMEM((_CW, _D), jnp.float32),
        pltpu.VMEM((_CW, _D), jnp.float32),
        pltpu.VMEM_SHARED((_NH, _D), jnp.float32),
        pltpu.SemaphoreType.DMA,
        pltpu.SemaphoreType.DMA,
        pltpu.SemaphoreType.DMA,
        pltpu.SemaphoreType.DMA,
        pltpu.SemaphoreType.DMA,
        pltpu.SemaphoreType.DMA,
    ],
)
def _edge_kernel(z_hbm, src_hbm, dst_hbm, agg_hbm,
                 srcbuf, dstbuf, rows0, rows1, rows2, aggsh,
                 g0, g1, g2, s0, s1, s2):
    cid = lax.axis_index("c")
    tid = lax.axis_index("s")
    zeros = jnp.zeros((16,), jnp.float32)
    rows = (rows0, rows1, rows2)
    gsem = (g0, g1, g2)
    ssem = (s0, s1, s2)

    def wait_gather(k):
        pltpu.make_async_copy(z_hbm.at[srcbuf.at[0]], rows[k],
                              gsem[k]).wait()

    def wait_scatter(k):
        pltpu.make_async_copy(rows[k], aggsh.at[dstbuf.at[0]],
                              ssem[k]).wait()

    for ri in range(_R // 2):
        r = cid * (_R // 2) + ri
        roff = r * _NH

        def zb_body(j, carry):
            for l in range(8):
                rows0[j, pl.ds(l * 16, 16)] = zeros
            return carry

        lax.fori_loop(0, _CW, zb_body, 0)
        for p in range(_RPT // _CW):
            pltpu.sync_copy(
                rows0, aggsh.at[pl.ds(tid * _RPT + p * _CW, _CW), :])
        plsc.subcore_barrier()

        pltpu.sync_copy(src_hbm.at[r, pl.ds(tid * _CPT, _CPT), :],
                        srcbuf.at[pl.ds(0, _CPT), :])
        pltpu.sync_copy(dst_hbm.at[r, pl.ds(tid * _CPT, _CPT), :], dstbuf)

        def off_body(j, carry):
            for l in range(_CW // 16):
                sl = pl.ds(l * 16, 16)
                srcbuf[j, sl] = srcbuf[j, sl] + roff
            return carry

        lax.fori_loop(0, _CPT, off_body, 0)
        pad = roff + _N + lax.iota(jnp.int32, 16)
        for j in (_CPT, _CPT + 1):
            for l in range(_CW // 16):
                srcbuf[j, pl.ds(l * 16, 16)] = pad + l * 16

        def step(c, k, first):
            wait_gather(k)
            pltpu.async_copy(rows[k], aggsh.at[dstbuf.at[c]], ssem[k],
                             add=True)
            k2 = (k + 2) % 3
            if not first:
                wait_scatter(k2)
            pltpu.async_copy(z_hbm.at[srcbuf.at[c + 2]], rows[k2], gsem[k2])

        pltpu.async_copy(z_hbm.at[srcbuf.at[0]], rows[0], gsem[0])
        pltpu.async_copy(z_hbm.at[srcbuf.at[1]], rows[1], gsem[1])
        step(0, 0, True)
        step(1, 1, False)
        step(2, 2, False)

        def chunk_body(jj, carry):
            c = jj * 3
            step(c, 0, False)
            step(c + 1, 1, False)
            step(c + 2, 2, False)
            return carry

        lax.fori_loop(1, 21, chunk_body, 0)
        step(63, 0, False)
        wait_scatter((_CPT - 1) % 3)
        wait_gather(_CPT % 3)
        wait_gather((_CPT + 1) % 3)
        plsc.subcore_barrier()
        pltpu.sync_copy(aggsh.at[pl.ds(tid * _RPT, _RPT), :],
                        agg_hbm.at[r, pl.ds(tid * _RPT, _RPT), :])
        plsc.subcore_barrier()


_BLK2 = 1024


def _out_body(agg_ref, cnt_ref, b_ref, o_ref):
    acc = jnp.broadcast_to(jnp.sum(b_ref[...], axis=0)[None, :], (_BLK2, _D))
    for r in range(_R):
        cnt = jnp.sum(cnt_ref[r, 0], axis=0)
        norm = lax.rsqrt(jnp.maximum(cnt, 1.0))
        acc = acc + agg_ref[r] * norm[:, None]
    o_ref[...] = acc


def _finalize(agg, cnt, b):
    return pl.pallas_call(
        _out_body,
        grid=(_NH // _BLK2,),
        in_specs=[
            pl.BlockSpec((_R, _BLK2, _D), lambda i: (0, i, 0)),
            pl.BlockSpec((_R, 1, _TILES, _BLK2), lambda i: (0, 1, 0, i)),
            pl.BlockSpec((_R, _D), lambda i: (0, 0)),
        ],
        out_specs=pl.BlockSpec((_BLK2, _D), lambda i: (i, 0)),
        out_shape=jax.ShapeDtypeStruct((_NH, _D), jnp.float32),
    )(agg, cnt, b)


def kernel(features, edge_index, W, b):
    x_pad = jnp.concatenate(
        [features, jnp.zeros((_NH - _N, _D), jnp.float32)], axis=0)
    fill = _N + (jnp.arange(_EPAD - _E, dtype=jnp.int32) % 128)
    fill = jnp.broadcast_to(fill[None, :], (_R, _EPAD - _E))
    src_pad = jnp.concatenate(
        [edge_index[:, 0, :], fill], axis=1).reshape(_R, _NCHUNK, _CW)
    dst_pad = jnp.concatenate(
        [edge_index[:, 1, :], fill], axis=1).reshape(_R, _NCHUNK, _CW)
    ei_pad = jnp.stack([src_pad, dst_pad], axis=1)

    cnt = _degree_kernel(ei_pad)
    z = _zmm(x_pad, cnt, W)
    agg = _edge_kernel(z.reshape(_R * _NH, _D), src_pad, dst_pad)
    return _finalize(agg, cnt, b)[:_N]
